# baseline (speedup 1.0000x reference)
import functools

import jax
import jax.numpy as jnp
import numpy as np
from jax.experimental import pallas as pl
from jax.experimental.pallas import tpu as pltpu
from jax.experimental.pallas import tpu_sc as plsc

B, AT, NBR = 8, 512, 64
DN, DE = 128, 128
NL = 3
GF_END = 8.0
NA = B * AT
E = NA * NBR
TA = 64
NBLK = NA // TA
G = 2
BG = B // G
S = NA // G
EG = E // G
W = 128
H = DN // 2

_BF = jnp.bfloat16
_HMASK = -65536
_RND = 0x8000


@functools.cache
def _mesh():
  return plsc.VectorSubcoreMesh(core_axis_name="c", subcore_axis_name="s")


def _sc_gather(src, idx2, n, d):
  @functools.partial(
      pl.kernel,
      out_type=jax.ShapeDtypeStruct((n, d), src.dtype),
      mesh=_mesh(),
  )
  def k(src_hbm, idx_hbm, out_hbm):
    def body(i_vmem, o_vmem):
      pltpu.sync_copy(src_hbm.at[i_vmem.at[0]], o_vmem)

    pltpu.emit_pipeline(
        body,
        grid=(n // W,),
        in_specs=[pl.BlockSpec((1, W), lambda i: (0, i))],
        out_specs=[pl.BlockSpec((W, d), lambda i: (i, 0))],
        core_axis_name=("c", "s"),
        dimension_semantics=(pltpu.PARALLEL,),
    )(idx_hbm, out_hbm)

  return k(src, idx2)


def _i32(x):
  return jax.lax.bitcast_convert_type(x, jnp.int32)


def _f32(x):
  return jax.lax.bitcast_convert_type(x, jnp.float32)


def _pk(lo, hi):
  lw = jnp.right_shift(_i32(lo) + _RND, 16) & 0xFFFF
  hw = (_i32(hi) + _RND) & _HMASK
  return lw | hw


def _uplo(w):
  return _f32(jnp.left_shift(w, 16))


def _uphi(w):
  return _f32(w & _HMASK)


def _dot(x, w_ref):
  return jax.lax.dot_general(
      x.astype(_BF), w_ref[...],
      (((1,), (0,)), ((), ())),
      preferred_element_type=jnp.float32)


def _t0p_body(node0_ref, posp_ref, wn_ref, bn_ref, tp_ref):
  t0 = jnp.tanh(_dot(node0_ref[...], wn_ref) + bn_ref[...])
  tw = _pk(t0[:, :H], t0[:, H:])
  posw = _i32(posp_ref[...])
  pad = jnp.zeros((TA, DN - H - 16), jnp.int32)
  tp_ref[...] = jnp.concatenate([tw, posw, pad], axis=1)


def _edge0(r2_3, coeff_ref):
  return jnp.exp(r2_3 * coeff_ref[...][0][None, None, :])


def _dc0_body(g0_ref, posp_ref, mask3_ref, node_ref, coeff_ref,
              we_ref, be_ref, wu_ref, wn_ref, bn_ref,
              r2_ref, uv_ref, n_ref, nt_ref):
  pnb = _f32(g0_ref[..., H:H + 16])
  dist = pnb - posp_ref[...][:, None, :]
  d2 = dist * dist
  r23 = jnp.sum(d2, axis=-1, keepdims=True)
  m3 = mask3_ref[...]
  r2m3 = jnp.where(m3 > 0, jnp.maximum(r23, 1e-12), 0.0)
  inv = jnp.where(r2m3 > 0, 1.0 / jnp.maximum(jnp.sqrt(r2m3), 1e-9), 0.0)
  uv_ref[...] = dist * inv
  r2_ref[...] = r2m3
  tw = g0_ref[..., :H]
  e0 = _edge0(r2m3, coeff_ref).reshape(TA * NBR, DN)
  e2 = (_dot(e0, we_ref) + be_ref[...]).reshape(TA, NBR, DN)
  m_lo = _uplo(tw) * e2[..., :H] * m3
  m_hi = _uphi(tw) * e2[..., H:] * m3
  agg = jnp.concatenate(
      [jnp.sum(m_lo, axis=1), jnp.sum(m_hi, axis=1)], axis=-1)
  node2 = node_ref[...] + _dot(agg, wu_ref)
  n_ref[...] = node2
  nt_ref[...] = _pk(node2, jnp.tanh(_dot(node2, wn_ref) + bn_ref[...]))


def _p_body(first, emit_t, e_ref, coeff_ref, gnt_ref, node_ref, mask3_ref,
            wp_ref, we_ref, be_ref, wu_ref, wn_ref, bn_ref,
            eo_ref, no_ref, nto_ref):
  node = node_ref[...]
  w = gnt_ref[...]
  gn = _uplo(w)
  gt = _uphi(w)
  pair = (node[:, None, :] * gn).reshape(TA * NBR, DN)
  if first:
    e_base = _edge0(e_ref[...], coeff_ref)
  else:
    ew = e_ref[...]
    e_base = jnp.concatenate([_uplo(ew), _uphi(ew)], axis=-1)
  enew = e_base + _dot(pair, wp_ref).reshape(TA, NBR, DN)
  eo_ref[...] = _pk(enew[..., :H], enew[..., H:])
  e2 = (_dot(enew.reshape(TA * NBR, DN), we_ref) + be_ref[...])
  m = gt * e2.reshape(TA, NBR, DN) * mask3_ref[...]
  agg = jnp.sum(m, axis=1)
  node2 = node + _dot(agg, wu_ref)
  no_ref[...] = node2
  if emit_t:
    nto_ref[...] = _pk(node2, jnp.tanh(_dot(node2, wn_ref) + bn_ref[...]))
  else:
    nto_ref[...] = jnp.zeros((TA, DN), jnp.int32)


def _d2_body(e_ref, gn_ref, node_ref, wp_ref, eo_ref):
  pair = (node_ref[...][:, None, :] * gn_ref[...]).reshape(TA * NBR, DN)
  ew = e_ref[...]
  e_base = jnp.concatenate([_uplo(ew), _uphi(ew)], axis=-1)
  eo_ref[...] = e_base + _dot(pair, wp_ref).reshape(TA, NBR, DN)


s_edge = pl.BlockSpec((TA, NBR, DN), lambda i: (i, 0, 0))
s_e64 = pl.BlockSpec((TA, NBR, H), lambda i: (i, 0, 0))
s_r23 = pl.BlockSpec((TA, NBR, 1), lambda i: (i, 0, 0))
s_mask3 = pl.BlockSpec((TA, NBR, 1), lambda i: (i, 0, 0))
s_node = pl.BlockSpec((TA, DN), lambda i: (i, 0))
s_w = pl.BlockSpec((DN, DN), lambda i: (0, 0))
s_b = pl.BlockSpec((1, DN), lambda i: (0, 0))
s_pos = pl.BlockSpec((TA, 16), lambda i: (i, 0))
s_uv = pl.BlockSpec((TA, NBR, 16), lambda i: (i, 0, 0))


def _tc_call(body, in_specs, out_specs, out_shape, nblk=NBLK):
  return pl.pallas_call(
      body,
      grid=(nblk,),
      in_specs=in_specs,
      out_specs=out_specs,
      out_shape=out_shape,
      compiler_params=pltpu.CompilerParams(
          dimension_semantics=("arbitrary",)),
  )


def kernel(positions, cell, cell_offset, neighbor_mask, atom_mask,
           node_table, Wn, bn, We, be, Wu, Wp,
           atomic_numbers, nbr_idx):
  del cell, cell_offset, atom_mask
  f32 = jnp.float32
  i32 = jnp.int32

  nbr_idx = nbr_idx.astype(i32)
  idxl = (nbr_idx
          + ((jnp.arange(B, dtype=i32) % BG) * AT)[:, None, None])
  idxl = idxl.reshape(G, 1, EG)
  an2 = atomic_numbers.astype(i32).reshape(1, NA)
  posp = jnp.pad(positions.reshape(NA, 3).astype(f32), ((0, 0), (0, 13)))
  maskp = neighbor_mask.reshape(NA, NBR).astype(f32)
  mask3 = maskp.reshape(NA, NBR, 1)
  offsets = np.linspace(0.0, GF_END, DE)
  widths = np.maximum(offsets, GF_END / DE)
  coeff = jnp.asarray(-0.5 / (widths * widths), f32).reshape(1, DE)
  wn = [Wn[l].astype(_BF) for l in range(NL)]
  we = [We[l].astype(_BF) for l in range(NL)]
  wu = [Wu[l].astype(_BF) for l in range(NL)]
  wp = [Wp[l].astype(_BF) for l in range(NL)]
  bnl = [bn[l].reshape(1, DN).astype(f32) for l in range(NL)]
  bel = [be[l].reshape(1, DN).astype(f32) for l in range(NL)]

  node0 = _sc_gather(node_table.astype(f32), an2, NA, DN)

  tp0 = _tc_call(
      _t0p_body,
      [s_node, s_pos, s_w, s_b],
      s_node,
      jax.ShapeDtypeStruct((NA, DN), i32),
  )(node0, posp, wn[0], bnl[0])

  nb = S // TA
  edges, uvs = [], []
  for g in range(G):
    sl = slice(g * S, (g + 1) * S)
    idx2 = idxl[g]
    pos_g, mask_g = posp[sl], mask3[sl]

    g0 = _sc_gather(tp0[sl], idx2, EG, DN).reshape(S, NBR, DN)

    r2m, uv, node1, nt1 = _tc_call(
        _dc0_body,
        [s_edge, s_pos, s_mask3, s_node, s_b, s_w, s_b, s_w, s_w, s_b],
        [s_r23, s_uv, s_node, s_node],
        (jax.ShapeDtypeStruct((S, NBR, 1), f32),
         jax.ShapeDtypeStruct((S, NBR, 16), f32),
         jax.ShapeDtypeStruct((S, DN), f32),
         jax.ShapeDtypeStruct((S, DN), i32)),
        nblk=nb,
    )(g0, pos_g, mask_g, node0[sl], coeff, we[0], bel[0], wu[0], wn[1],
      bnl[1])

    gnt1 = _sc_gather(nt1, idx2, EG, DN).reshape(S, NBR, DN)
    edge1, node2, nt2 = _tc_call(
        functools.partial(_p_body, True, True),
        [s_r23, s_b, s_edge, s_node, s_mask3, s_w, s_w, s_b, s_w, s_w,
         s_b],
        [s_e64, s_node, s_node],
        (jax.ShapeDtypeStruct((S, NBR, H), i32),
         jax.ShapeDtypeStruct((S, DN), f32),
         jax.ShapeDtypeStruct((S, DN), i32)),
        nblk=nb,
    )(r2m, coeff, gnt1, node1, mask_g, wp[0], we[1], bel[1], wu[1],
      wn[2], bnl[2])

    gnt2 = _sc_gather(nt2, idx2, EG, DN).reshape(S, NBR, DN)
    edge2, node3, _ = _tc_call(
        functools.partial(_p_body, False, False),
        [s_e64, s_b, s_edge, s_node, s_mask3, s_w, s_w, s_b, s_w, s_w,
         s_b],
        [s_e64, s_node, s_node],
        (jax.ShapeDtypeStruct((S, NBR, H), i32),
         jax.ShapeDtypeStruct((S, DN), f32),
         jax.ShapeDtypeStruct((S, DN), i32)),
        nblk=nb,
    )(edge1, coeff, gnt2, node2, mask_g, wp[1], we[2], bel[2], wu[2],
      wn[2], bnl[2])

    gn3 = _sc_gather(node3, idx2, EG, DN).reshape(S, NBR, DN)
    edge3 = _tc_call(
        _d2_body,
        [s_e64, s_edge, s_node, s_w],
        s_edge,
        jax.ShapeDtypeStruct((S, NBR, DN), f32),
        nblk=nb,
    )(edge2, gn3, node3, wp[2])
    edges.append(edge3)
    uvs.append(uv)

  edge3 = jnp.concatenate(edges, axis=0)
  uv = jnp.concatenate(uvs, axis=0)
  return (edge3.reshape(B, AT, NBR, DE),
          uv.reshape(B, AT, NBR, 16)[..., :3])


# speedup vs baseline: 1.1489x; 1.1489x over previous
import functools

import jax
import jax.numpy as jnp
import numpy as np
from jax.experimental import pallas as pl
from jax.experimental.pallas import tpu as pltpu
from jax.experimental.pallas import tpu_sc as plsc

B, AT, NBR = 8, 512, 64
DN, DE = 128, 128
NL = 3
GF_END = 8.0
NA = B * AT
E = NA * NBR
TA = 64
NBLK = NA // TA
W = 256
H = DN // 2

_BF = jnp.bfloat16
_HMASK = -65536
_RND = 0x8000


@functools.cache
def _mesh():
  return plsc.VectorSubcoreMesh(core_axis_name="c", subcore_axis_name="s")


def _sc_gather(src, idx2, n, d):
  @functools.partial(
      pl.kernel,
      out_type=jax.ShapeDtypeStruct((n, d), src.dtype),
      mesh=_mesh(),
  )
  def k(src_hbm, idx_hbm, out_hbm):
    def body(i_vmem, o_vmem):
      pltpu.sync_copy(src_hbm.at[i_vmem.at[0]], o_vmem)

    pltpu.emit_pipeline(
        body,
        grid=(n // W,),
        in_specs=[pl.BlockSpec((1, W), lambda i: (0, i))],
        out_specs=[pl.BlockSpec((W, d), lambda i: (i, 0))],
        core_axis_name=("c", "s"),
        dimension_semantics=(pltpu.PARALLEL,),
    )(idx_hbm, out_hbm)

  return k(src, idx2)


def _i32(x):
  return jax.lax.bitcast_convert_type(x, jnp.int32)


def _f32(x):
  return jax.lax.bitcast_convert_type(x, jnp.float32)


def _pk(lo, hi):
  lw = jnp.right_shift(_i32(lo) + _RND, 16) & 0xFFFF
  hw = (_i32(hi) + _RND) & _HMASK
  return lw | hw


def _uplo(w):
  return _f32(jnp.left_shift(w, 16))


def _uphi(w):
  return _f32(w & _HMASK)


def _dot(x, w_ref):
  return jax.lax.dot_general(
      x.astype(_BF), w_ref[...],
      (((1,), (0,)), ((), ())),
      preferred_element_type=jnp.float32)


def _t0p_body(node0_ref, posp_ref, wn_ref, bn_ref, tp_ref):
  t0 = jnp.tanh(_dot(node0_ref[...], wn_ref) + bn_ref[...])
  tw = _pk(t0[:, :H], t0[:, H:])
  posw = _i32(posp_ref[...])
  pad = jnp.zeros((TA, DN - H - 16), jnp.int32)
  tp_ref[...] = jnp.concatenate([tw, posw, pad], axis=1)


def _edge0(r2_3, coeff_ref):
  return jnp.exp(r2_3 * coeff_ref[...][0][None, None, :])


def _dc0_body(g0_ref, posp_ref, mask3_ref, node_ref, coeff_ref,
              we_ref, be_ref, wu_ref, wn_ref, bn_ref,
              r2_ref, uv_ref, n_ref, nt_ref):
  pnb = _f32(g0_ref[..., H:H + 16])
  dist = pnb - posp_ref[...][:, None, :]
  d2 = dist * dist
  r23 = jnp.sum(d2, axis=-1, keepdims=True)
  m3 = mask3_ref[...]
  r2m3 = jnp.where(m3 > 0, jnp.maximum(r23, 1e-12), 0.0)
  inv = jnp.where(r2m3 > 0, 1.0 / jnp.maximum(jnp.sqrt(r2m3), 1e-9), 0.0)
  uv_ref[...] = dist * inv
  r2_ref[...] = r2m3
  tw = g0_ref[..., :H]
  e0 = _edge0(r2m3, coeff_ref).reshape(TA * NBR, DN)
  e2 = (_dot(e0, we_ref) + be_ref[...]).reshape(TA, NBR, DN)
  m_lo = _uplo(tw) * e2[..., :H] * m3
  m_hi = _uphi(tw) * e2[..., H:] * m3
  agg = jnp.concatenate(
      [jnp.sum(m_lo, axis=1), jnp.sum(m_hi, axis=1)], axis=-1)
  node2 = node_ref[...] + _dot(agg, wu_ref)
  n_ref[...] = node2
  nt_ref[...] = _pk(node2, jnp.tanh(_dot(node2, wn_ref) + bn_ref[...]))


def _p_body(first, emit_t, e_ref, coeff_ref, gnt_ref, node_ref, mask3_ref,
            wp_ref, we_ref, be_ref, wu_ref, wn_ref, bn_ref,
            eo_ref, no_ref, nto_ref):
  node = node_ref[...]
  w = gnt_ref[...]
  gn = _uplo(w)
  gt = _uphi(w)
  pair = (node[:, None, :] * gn).reshape(TA * NBR, DN)
  if first:
    e_base = _edge0(e_ref[...], coeff_ref)
  else:
    ew = e_ref[...]
    e_base = jnp.concatenate([_uplo(ew), _uphi(ew)], axis=-1)
  enew = e_base + _dot(pair, wp_ref).reshape(TA, NBR, DN)
  eo_ref[...] = _pk(enew[..., :H], enew[..., H:])
  e2 = (_dot(enew.reshape(TA * NBR, DN), we_ref) + be_ref[...])
  m = gt * e2.reshape(TA, NBR, DN) * mask3_ref[...]
  agg = jnp.sum(m, axis=1)
  node2 = node + _dot(agg, wu_ref)
  no_ref[...] = node2
  if emit_t:
    nto_ref[...] = _pk(node2, jnp.tanh(_dot(node2, wn_ref) + bn_ref[...]))
  else:
    nto_ref[...] = jnp.zeros((TA, DN), jnp.int32)


def _d2_body(e_ref, gn_ref, node_ref, wp_ref, eo_ref):
  pair = (node_ref[...][:, None, :] * gn_ref[...]).reshape(TA * NBR, DN)
  ew = e_ref[...]
  e_base = jnp.concatenate([_uplo(ew), _uphi(ew)], axis=-1)
  eo_ref[...] = e_base + _dot(pair, wp_ref).reshape(TA, NBR, DN)


s_edge = pl.BlockSpec((TA, NBR, DN), lambda i: (i, 0, 0))
s_e64 = pl.BlockSpec((TA, NBR, H), lambda i: (i, 0, 0))
s_r23 = pl.BlockSpec((TA, NBR, 1), lambda i: (i, 0, 0))
s_mask3 = pl.BlockSpec((TA, NBR, 1), lambda i: (i, 0, 0))
s_node = pl.BlockSpec((TA, DN), lambda i: (i, 0))
s_w = pl.BlockSpec((DN, DN), lambda i: (0, 0))
s_b = pl.BlockSpec((1, DN), lambda i: (0, 0))
s_pos = pl.BlockSpec((TA, 16), lambda i: (i, 0))
s_uv = pl.BlockSpec((TA, NBR, 16), lambda i: (i, 0, 0))


def _tc_call(body, in_specs, out_specs, out_shape):
  return pl.pallas_call(
      body,
      grid=(NBLK,),
      in_specs=in_specs,
      out_specs=out_specs,
      out_shape=out_shape,
      compiler_params=pltpu.CompilerParams(
          dimension_semantics=("arbitrary",)),
  )


def kernel(positions, cell, cell_offset, neighbor_mask, atom_mask,
           node_table, Wn, bn, We, be, Wu, Wp,
           atomic_numbers, nbr_idx):
  del cell, cell_offset, atom_mask
  f32 = jnp.float32
  i32 = jnp.int32

  nbr_idx = nbr_idx.astype(i32)
  idxg = (nbr_idx + (jnp.arange(B, dtype=i32) * AT)[:, None, None])
  idx2 = idxg.reshape(1, E)
  an2 = atomic_numbers.astype(i32).reshape(1, NA)
  posp = jnp.pad(positions.reshape(NA, 3).astype(f32), ((0, 0), (0, 13)))
  maskp = neighbor_mask.reshape(NA, NBR).astype(f32)
  mask3 = maskp.reshape(NA, NBR, 1)
  offsets = np.linspace(0.0, GF_END, DE)
  widths = np.maximum(offsets, GF_END / DE)
  coeff = jnp.asarray(-0.5 / (widths * widths), f32).reshape(1, DE)
  wn = [Wn[l].astype(_BF) for l in range(NL)]
  we = [We[l].astype(_BF) for l in range(NL)]
  wu = [Wu[l].astype(_BF) for l in range(NL)]
  wp = [Wp[l].astype(_BF) for l in range(NL)]
  bnl = [bn[l].reshape(1, DN).astype(f32) for l in range(NL)]
  bel = [be[l].reshape(1, DN).astype(f32) for l in range(NL)]

  node0 = _sc_gather(node_table.astype(f32), an2, NA, DN)

  tp0 = _tc_call(
      _t0p_body,
      [s_node, s_pos, s_w, s_b],
      s_node,
      jax.ShapeDtypeStruct((NA, DN), i32),
  )(node0, posp, wn[0], bnl[0])

  g0 = _sc_gather(tp0, idx2, E, DN).reshape(NA, NBR, DN)

  r2m, uv, node1, nt1 = _tc_call(
      _dc0_body,
      [s_edge, s_pos, s_mask3, s_node, s_b, s_w, s_b, s_w, s_w, s_b],
      [s_r23, s_uv, s_node, s_node],
      (jax.ShapeDtypeStruct((NA, NBR, 1), f32),
       jax.ShapeDtypeStruct((NA, NBR, 16), f32),
       jax.ShapeDtypeStruct((NA, DN), f32),
       jax.ShapeDtypeStruct((NA, DN), i32)),
  )(g0, posp, mask3, node0, coeff, we[0], bel[0], wu[0], wn[1], bnl[1])

  gnt1 = _sc_gather(nt1, idx2, E, DN).reshape(NA, NBR, DN)
  edge1, node2, nt2 = _tc_call(
      functools.partial(_p_body, True, True),
      [s_r23, s_b, s_edge, s_node, s_mask3, s_w, s_w, s_b, s_w, s_w, s_b],
      [s_e64, s_node, s_node],
      (jax.ShapeDtypeStruct((NA, NBR, H), i32),
       jax.ShapeDtypeStruct((NA, DN), f32),
       jax.ShapeDtypeStruct((NA, DN), i32)),
  )(r2m, coeff, gnt1, node1, mask3, wp[0], we[1], bel[1], wu[1], wn[2],
    bnl[2])

  gnt2 = _sc_gather(nt2, idx2, E, DN).reshape(NA, NBR, DN)
  edge2, node3, _ = _tc_call(
      functools.partial(_p_body, False, False),
      [s_e64, s_b, s_edge, s_node, s_mask3, s_w, s_w, s_b, s_w, s_w, s_b],
      [s_e64, s_node, s_node],
      (jax.ShapeDtypeStruct((NA, NBR, H), i32),
       jax.ShapeDtypeStruct((NA, DN), f32),
       jax.ShapeDtypeStruct((NA, DN), i32)),
  )(edge1, coeff, gnt2, node2, mask3, wp[1], we[2], bel[2], wu[2], wn[2],
    bnl[2])

  gn3 = _sc_gather(node3, idx2, E, DN).reshape(NA, NBR, DN)
  edge3 = _tc_call(
      _d2_body,
      [s_e64, s_edge, s_node, s_w],
      s_edge,
      jax.ShapeDtypeStruct((NA, NBR, DN), f32),
  )(edge2, gn3, node3, wp[2])

  return (edge3.reshape(B, AT, NBR, DE),
          uv.reshape(B, AT, NBR, 16)[..., :3])
